# baseline (speedup 1.0000x reference)
import functools

import jax
import jax.numpy as jnp
from jax import lax
from jax.experimental import pallas as pl
from jax.experimental.pallas import tpu as pltpu
from jax.experimental.pallas import tpu_sc as plsc

_N = 10000
_E = 160000
_IN = 128
_H = 8
_ED = 16
_OUT = 32

_NC = 2
_NS = 16
_NW = _NC * _NS
_PER_W = _E // _NW
_GCH = 128
_GNC = 40
_GTAIL = _PER_W - (_GNC - 1) * _GCH
_SCH = 104
_SNC = 49
_STAIL = _PER_W - (_SNC - 1) * _SCH
_ZCH = 40
_RS = 640
_RS_LAST = _N - (_NS - 1) * _RS
_MW = 128


def _sc_mesh():
    return plsc.VectorSubcoreMesh(core_axis_name="c", subcore_axis_name="s")


def _make_sc_gather(width):
    nb = 4

    @functools.partial(
        pl.kernel,
        out_type=jax.ShapeDtypeStruct((_E, width), jnp.float32),
        mesh=_sc_mesh(),
        scratch_types=[
            pltpu.VMEM((_GNC, _GCH), jnp.int32),
        ] + [pltpu.VMEM((_GCH, width), jnp.float32) for _ in range(nb)]
        + [pltpu.SemaphoreType.DMA for _ in range(2 * nb)],
    )
    def g(table_hbm, idx_hbm, out_hbm, idx_v, *bs):
        bufs = bs[:nb]
        gsems = bs[nb:2 * nb]
        osems = bs[2 * nb:]
        wid = lax.axis_index("s") * _NC + lax.axis_index("c")
        base = wid * _PER_W
        pltpu.sync_copy(idx_hbm.at[wid], idx_v)

        def out_slice(j):
            rows = _GCH if j < _GNC - 1 else _GTAIL
            return (bufs[j % nb] if rows == _GCH
                    else bufs[j % nb].at[pl.ds(0, rows)],
                    out_hbm.at[pl.ds(base + j * _GCH, rows)])

        pltpu.async_copy(table_hbm.at[idx_v.at[0]], bufs[0], gsems[0])
        pltpu.async_copy(table_hbm.at[idx_v.at[1]], bufs[1], gsems[1])
        for j in range(_GNC):
            b = j % nb
            nj = j + 2
            if nj < _GNC:
                if nj - nb >= 0:
                    src, dst = out_slice(nj - nb)
                    pltpu.make_async_copy(src, dst, osems[nj % nb]).wait()
                pltpu.async_copy(table_hbm.at[idx_v.at[nj]],
                                 bufs[nj % nb], gsems[nj % nb])
            pltpu.make_async_copy(
                table_hbm.at[idx_v.at[j]], bufs[b], gsems[b]).wait()
            src, dst = out_slice(j)
            pltpu.async_copy(src, dst, osems[b])
        for j in range(_GNC - nb, _GNC):
            src, dst = out_slice(j)
            pltpu.make_async_copy(src, dst, osems[j % nb]).wait()

    return g


@functools.cache
def _build_sc_scatter_add():
    @functools.partial(
        pl.kernel,
        out_type=jax.ShapeDtypeStruct((_NC, _N, _MW), jnp.float32),
        mesh=_sc_mesh(),
        scratch_types=[
            pltpu.VMEM((_SNC, _SCH), jnp.int32),
            pltpu.VMEM((_SCH, _MW), jnp.float32),
            pltpu.VMEM((_SCH, _MW), jnp.float32),
            pltpu.VMEM_SHARED((_N + 8, _MW), jnp.float32),
            pltpu.SemaphoreType.DMA,
            pltpu.SemaphoreType.DMA,
        ],
    )
    def k(z_hbm, msg_hbm, idx_hbm, out_hbm, idx_v, b0, b1, acc, s0, s1):
        c = lax.axis_index("c")
        s = lax.axis_index("s")
        wid = s * _NC + c
        base = wid * _PER_W
        bufs = (b0, b1)
        sems = (s0, s1)

        pltpu.sync_copy(z_hbm, b0.at[pl.ds(0, _ZCH)])

        nz = lax.select(s == _NS - 1, _RS_LAST // _ZCH, _RS // _ZCH)

        def zcopy(j, carry):
            pltpu.sync_copy(b0.at[pl.ds(0, _ZCH)],
                            acc.at[pl.ds(s * _RS + j * _ZCH, _ZCH)])
            return carry

        lax.fori_loop(0, nz, zcopy, 0)
        pltpu.sync_copy(idx_hbm.at[wid], idx_v)
        plsc.subcore_barrier()

        def start_load(j, b):
            if j < _SNC - 1:
                pltpu.async_copy(msg_hbm.at[pl.ds(base + j * _SCH, _SCH)],
                                 bufs[b], sems[b])
            else:
                pltpu.async_copy(
                    msg_hbm.at[pl.ds(base + j * _SCH, _STAIL)],
                    bufs[b].at[pl.ds(0, _STAIL)], sems[b])

        def wait_load(j, b):
            if j < _SNC - 1:
                pltpu.make_async_copy(
                    msg_hbm.at[pl.ds(base + j * _SCH, _SCH)],
                    bufs[b], sems[b]).wait()
            else:
                pltpu.make_async_copy(
                    msg_hbm.at[pl.ds(base + j * _SCH, _STAIL)],
                    bufs[b].at[pl.ds(0, _STAIL)], sems[b]).wait()

        start_load(0, 0)
        start_load(1, 1)
        for j in range(_SNC):
            b = j % 2
            wait_load(j, b)
            pltpu.sync_copy(bufs[b], acc.at[idx_v.at[j]], add=True)
            if j + 2 < _SNC:
                start_load(j + 2, b)
        plsc.subcore_barrier()

        @pl.when(s < _NS - 1)
        def _copy_main():
            pltpu.sync_copy(acc.at[pl.ds(s * _RS, _RS)],
                            out_hbm.at[c, pl.ds(s * _RS, _RS)])

        @pl.when(s == _NS - 1)
        def _copy_last():
            pltpu.sync_copy(acc.at[pl.ds((_NS - 1) * _RS, _RS_LAST)],
                            out_hbm.at[c, pl.ds((_NS - 1) * _RS, _RS_LAST)])

    return k


def _sc_scatter_add(msg, idx3, zrows):
    return _build_sc_scatter_add()(zrows, msg, idx3)


_B = 1000


def _tc_edge1_body(ea_ref, xg_ref, w1_ref, b1_ref, w2t_ref, b2t_ref,
                   v1_ref, c1_ref, v2t_ref, c2t_ref, s1_ref, msg_ref, wm2_ref):
    ea = ea_ref[...]
    p1 = jnp.maximum(
        jnp.dot(ea, w1_ref[...], preferred_element_type=jnp.float32)
        + b1_ref[...], 0.0)
    wm1 = jnp.dot(p1, w2t_ref[...],
                  preferred_element_type=jnp.float32) + b2t_ref[...]
    xg = xg_ref[...]
    xgt = jnp.concatenate([xg] * _H, axis=1)
    msg = jnp.dot(wm1 * xgt, s1_ref[...],
                  preferred_element_type=jnp.float32)
    ones = jnp.ones((_B, 1), jnp.float32)
    zer = jnp.zeros((_B, _MW - _H - 1), jnp.float32)
    msg_ref[...] = jnp.concatenate([msg, ones, zer], axis=1)

    p2 = jnp.maximum(
        jnp.dot(ea, v1_ref[...], preferred_element_type=jnp.float32)
        + c1_ref[...], 0.0)
    wm2_ref[...] = jnp.dot(p2, v2t_ref[...],
                           preferred_element_type=jnp.float32) + c2t_ref[...]


def _tc_edge2_body(hg_ref, wm2_ref, s2_ref, msg_ref):
    hg8 = hg_ref[:, :_H]
    hgt = jnp.concatenate([hg8] * _H, axis=1)
    msg = jnp.dot(wm2_ref[...] * hgt, s2_ref[...],
                  preferred_element_type=jnp.float32)
    msg_ref[...] = jnp.concatenate(
        [msg, jnp.zeros((_B, _MW - _H), jnp.float32)], axis=1)


def _tc_node1_body(p_ref, x_ref, r1_ref, b1_ref, h_ref):
    ssum = p_ref[0] + p_ref[1]
    inv = 1.0 / jnp.maximum(ssum[:, _H:_H + 1], 1.0)
    mean = ssum[:, :_H] * inv
    xr = jnp.dot(x_ref[...], r1_ref[...], preferred_element_type=jnp.float32)
    h = jnp.maximum(mean + xr + b1_ref[...], 0.0)
    h_ref[...] = jnp.concatenate(
        [h, inv, jnp.zeros((_N, _MW - _H - 1), jnp.float32)], axis=1)


def _tc_out_body(q_ref, h_ref, r2_ref, b2_ref, ow_ref, ob_ref, o_ref):
    ssum = q_ref[0] + q_ref[1]
    mean = ssum[:, :_H] * h_ref[:, _H:_H + 1]
    g = mean + jnp.dot(h_ref[:, :_H], r2_ref[...],
                       preferred_element_type=jnp.float32) + b2_ref[...]
    o_ref[...] = jnp.dot(g, ow_ref[...],
                         preferred_element_type=jnp.float32) + ob_ref[...]


def _full(shape):
    return pl.BlockSpec(shape, lambda i: tuple(0 for _ in shape))


def kernel(x, edge_index, edge_attr, nn1_w1, nn1_b1, nn1_w2, nn1_b2, root1,
           bias1, nn2_w1, nn2_b1, nn2_w2, nn2_b2, root2, bias2, out_w, out_b):
    f32 = jnp.float32
    srcw = edge_index[0].reshape(_NW, _PER_W)
    src3 = jnp.concatenate(
        [srcw, jnp.zeros((_NW, _GNC * _GCH - _PER_W), jnp.int32)],
        axis=1).reshape(_NW, _GNC, _GCH)
    dstw = edge_index[1].reshape(_NW, _PER_W)
    dst3 = jnp.concatenate(
        [dstw, jnp.full((_NW, _SNC * _SCH - _PER_W), _N, jnp.int32)],
        axis=1).reshape(_NW, _SNC, _SCH)

    w2t1 = nn1_w2.reshape(128, _IN, _H).transpose(0, 2, 1).reshape(128, _H * _IN)
    b2t1 = nn1_b2.reshape(_IN, _H).T.reshape(1, _H * _IN)
    w2t2 = nn2_w2.reshape(128, _H, _H).transpose(0, 2, 1).reshape(128, _H * _H)
    b2t2 = nn2_b2.reshape(_H, _H).T.reshape(1, _H * _H)
    s1 = jnp.repeat(jnp.eye(_H, dtype=f32), _IN, axis=0)
    s2 = jnp.repeat(jnp.eye(_H, dtype=f32), _H, axis=0)
    b1r = nn1_b1.reshape(1, 128)
    c1r = nn2_b1.reshape(1, 128)
    bias1r = bias1.reshape(1, _H)
    bias2r = bias2.reshape(1, _H)
    out_br = out_b.reshape(1, _OUT)

    zrows = jnp.zeros((_ZCH, _MW), f32)

    xg = _make_sc_gather(_IN)(x, src3)

    grid = (_E // _B,)
    msg1, wm2 = pl.pallas_call(
        _tc_edge1_body,
        grid=grid,
        in_specs=[
            pl.BlockSpec((_B, _ED), lambda i: (i, 0)),
            pl.BlockSpec((_B, _IN), lambda i: (i, 0)),
            _full((_ED, 128)), _full((1, 128)),
            _full((128, _H * _IN)), _full((1, _H * _IN)),
            _full((_ED, 128)), _full((1, 128)),
            _full((128, _H * _H)), _full((1, _H * _H)),
            _full((_H * _IN, _H)),
        ],
        out_specs=[pl.BlockSpec((_B, _MW), lambda i: (i, 0)),
                   pl.BlockSpec((_B, _H * _H), lambda i: (i, 0))],
        out_shape=[jax.ShapeDtypeStruct((_E, _MW), f32),
                   jax.ShapeDtypeStruct((_E, _H * _H), f32)],
    )(edge_attr, xg, nn1_w1, b1r, w2t1, b2t1, nn2_w1, c1r, w2t2, b2t2, s1)

    p1 = _sc_scatter_add(msg1, dst3, zrows)

    hpad = pl.pallas_call(
        _tc_node1_body,
        out_shape=jax.ShapeDtypeStruct((_N, _MW), f32),
    )(p1, x, root1, bias1r)

    hg = _make_sc_gather(_MW)(hpad, src3)

    msg2 = pl.pallas_call(
        _tc_edge2_body,
        grid=grid,
        in_specs=[
            pl.BlockSpec((_B, _MW), lambda i: (i, 0)),
            pl.BlockSpec((_B, _H * _H), lambda i: (i, 0)),
            _full((_H * _H, _H)),
        ],
        out_specs=pl.BlockSpec((_B, _MW), lambda i: (i, 0)),
        out_shape=jax.ShapeDtypeStruct((_E, _MW), f32),
    )(hg, wm2, s2)

    p2 = _sc_scatter_add(msg2, dst3, zrows)

    out = pl.pallas_call(
        _tc_out_body,
        out_shape=jax.ShapeDtypeStruct((_N, _OUT), f32),
    )(p2, hpad, root2, bias2r, out_w, out_br)
    return out


# speedup vs baseline: 1.4358x; 1.4358x over previous
import functools

import jax
import jax.numpy as jnp
from jax import lax
from jax.experimental import pallas as pl
from jax.experimental.pallas import tpu as pltpu
from jax.experimental.pallas import tpu_sc as plsc

_N = 10000
_E = 160000
_IN = 128
_H = 8
_ED = 16
_OUT = 32

_NC = 2
_NS = 16
_NW = _NC * _NS
_PER_W = _E // _NW
_GCH = 64
_GNC = 79
_GTAIL = _PER_W - (_GNC - 1) * _GCH
_SCH = 104
_SNC = 49
_STAIL = _PER_W - (_SNC - 1) * _SCH
_ZCH = 40
_RS = 640
_RS_LAST = _N - (_NS - 1) * _RS
_MW = 128


def _sc_mesh():
    return plsc.VectorSubcoreMesh(core_axis_name="c", subcore_axis_name="s")


def _make_sc_gather(width):
    nb = 4

    @functools.partial(
        pl.kernel,
        out_type=jax.ShapeDtypeStruct((_E, width), jnp.float32),
        mesh=_sc_mesh(),
        scratch_types=[
            pltpu.VMEM((_GNC, _GCH), jnp.int32),
            pltpu.VMEM_SHARED((_N, width), jnp.float32),
        ] + [pltpu.VMEM((_GCH, width), jnp.float32) for _ in range(nb)]
        + [pltpu.SemaphoreType.DMA for _ in range(2 * nb)],
    )
    def g(table_hbm, idx_hbm, out_hbm, idx_v, tbl_s, *bs):
        bufs = bs[:nb]
        gsems = bs[nb:2 * nb]
        osems = bs[2 * nb:]
        s = lax.axis_index("s")
        wid = s * _NC + lax.axis_index("c")
        base = wid * _PER_W

        @pl.when(s < _NS - 1)
        def _stage_main():
            pltpu.sync_copy(table_hbm.at[pl.ds(s * _RS, _RS)],
                            tbl_s.at[pl.ds(s * _RS, _RS)])

        @pl.when(s == _NS - 1)
        def _stage_last():
            pltpu.sync_copy(table_hbm.at[pl.ds((_NS - 1) * _RS, _RS_LAST)],
                            tbl_s.at[pl.ds((_NS - 1) * _RS, _RS_LAST)])

        pltpu.sync_copy(idx_hbm.at[wid], idx_v)
        plsc.subcore_barrier()

        def out_slice(j):
            rows = _GCH if j < _GNC - 1 else _GTAIL
            return (bufs[j % nb] if rows == _GCH
                    else bufs[j % nb].at[pl.ds(0, rows)],
                    out_hbm.at[pl.ds(base + j * _GCH, rows)])

        pltpu.async_copy(tbl_s.at[idx_v.at[0]], bufs[0], gsems[0])
        pltpu.async_copy(tbl_s.at[idx_v.at[1]], bufs[1], gsems[1])
        for j in range(_GNC):
            b = j % nb
            nj = j + 2
            if nj < _GNC:
                if nj - nb >= 0:
                    src, dst = out_slice(nj - nb)
                    pltpu.make_async_copy(src, dst, osems[nj % nb]).wait()
                pltpu.async_copy(tbl_s.at[idx_v.at[nj]],
                                 bufs[nj % nb], gsems[nj % nb])
            pltpu.make_async_copy(
                tbl_s.at[idx_v.at[j]], bufs[b], gsems[b]).wait()
            src, dst = out_slice(j)
            pltpu.async_copy(src, dst, osems[b])
        for j in range(_GNC - nb, _GNC):
            src, dst = out_slice(j)
            pltpu.make_async_copy(src, dst, osems[j % nb]).wait()

    return g


@functools.cache
def _build_sc_scatter_add():
    @functools.partial(
        pl.kernel,
        out_type=jax.ShapeDtypeStruct((_NC, _N, _MW), jnp.float32),
        mesh=_sc_mesh(),
        scratch_types=[
            pltpu.VMEM((_SNC, _SCH), jnp.int32),
            pltpu.VMEM((_SCH, _MW), jnp.float32),
            pltpu.VMEM((_SCH, _MW), jnp.float32),
            pltpu.VMEM_SHARED((_N + 8, _MW), jnp.float32),
            pltpu.SemaphoreType.DMA,
            pltpu.SemaphoreType.DMA,
        ],
    )
    def k(z_hbm, msg_hbm, idx_hbm, out_hbm, idx_v, b0, b1, acc, s0, s1):
        c = lax.axis_index("c")
        s = lax.axis_index("s")
        wid = s * _NC + c
        base = wid * _PER_W
        bufs = (b0, b1)
        sems = (s0, s1)

        pltpu.sync_copy(z_hbm, b0.at[pl.ds(0, _ZCH)])

        nz = lax.select(s == _NS - 1, _RS_LAST // _ZCH, _RS // _ZCH)

        def zcopy(j, carry):
            pltpu.sync_copy(b0.at[pl.ds(0, _ZCH)],
                            acc.at[pl.ds(s * _RS + j * _ZCH, _ZCH)])
            return carry

        lax.fori_loop(0, nz, zcopy, 0)
        pltpu.sync_copy(idx_hbm.at[wid], idx_v)
        plsc.subcore_barrier()

        def start_load(j, b):
            if j < _SNC - 1:
                pltpu.async_copy(msg_hbm.at[pl.ds(base + j * _SCH, _SCH)],
                                 bufs[b], sems[b])
            else:
                pltpu.async_copy(
                    msg_hbm.at[pl.ds(base + j * _SCH, _STAIL)],
                    bufs[b].at[pl.ds(0, _STAIL)], sems[b])

        def wait_load(j, b):
            if j < _SNC - 1:
                pltpu.make_async_copy(
                    msg_hbm.at[pl.ds(base + j * _SCH, _SCH)],
                    bufs[b], sems[b]).wait()
            else:
                pltpu.make_async_copy(
                    msg_hbm.at[pl.ds(base + j * _SCH, _STAIL)],
                    bufs[b].at[pl.ds(0, _STAIL)], sems[b]).wait()

        start_load(0, 0)
        start_load(1, 1)
        for j in range(_SNC):
            b = j % 2
            wait_load(j, b)
            pltpu.sync_copy(bufs[b], acc.at[idx_v.at[j]], add=True)
            if j + 2 < _SNC:
                start_load(j + 2, b)
        plsc.subcore_barrier()

        @pl.when(s < _NS - 1)
        def _copy_main():
            pltpu.sync_copy(acc.at[pl.ds(s * _RS, _RS)],
                            out_hbm.at[c, pl.ds(s * _RS, _RS)])

        @pl.when(s == _NS - 1)
        def _copy_last():
            pltpu.sync_copy(acc.at[pl.ds((_NS - 1) * _RS, _RS_LAST)],
                            out_hbm.at[c, pl.ds((_NS - 1) * _RS, _RS_LAST)])

    return k


def _sc_scatter_add(msg, idx3, zrows):
    return _build_sc_scatter_add()(zrows, msg, idx3)


_B = 1000


def _tc_edge1_body(ea_ref, xg_ref, w1_ref, b1_ref, w2t_ref, b2t_ref,
                   v1_ref, c1_ref, v2t_ref, c2t_ref, s1_ref, msg_ref, wm2_ref):
    ea = ea_ref[...]
    p1 = jnp.maximum(
        jnp.dot(ea, w1_ref[...], preferred_element_type=jnp.float32)
        + b1_ref[...], 0.0)
    wm1 = jnp.dot(p1, w2t_ref[...],
                  preferred_element_type=jnp.float32) + b2t_ref[...]
    xg = xg_ref[...]
    xgt = jnp.concatenate([xg] * _H, axis=1)
    msg = jnp.dot(wm1 * xgt, s1_ref[...],
                  preferred_element_type=jnp.float32)
    ones = jnp.ones((_B, 1), jnp.float32)
    zer = jnp.zeros((_B, _MW - _H - 1), jnp.float32)
    msg_ref[...] = jnp.concatenate([msg, ones, zer], axis=1)

    p2 = jnp.maximum(
        jnp.dot(ea, v1_ref[...], preferred_element_type=jnp.float32)
        + c1_ref[...], 0.0)
    wm2_ref[...] = jnp.dot(p2, v2t_ref[...],
                           preferred_element_type=jnp.float32) + c2t_ref[...]


def _tc_edge2_body(hg_ref, wm2_ref, s2_ref, msg_ref):
    hg8 = hg_ref[:, :_H]
    hgt = jnp.concatenate([hg8] * _H, axis=1)
    msg = jnp.dot(wm2_ref[...] * hgt, s2_ref[...],
                  preferred_element_type=jnp.float32)
    msg_ref[...] = jnp.concatenate(
        [msg, jnp.zeros((_B, _MW - _H), jnp.float32)], axis=1)


def _tc_node1_body(p_ref, x_ref, r1_ref, b1_ref, h_ref):
    ssum = p_ref[0] + p_ref[1]
    inv = 1.0 / jnp.maximum(ssum[:, _H:_H + 1], 1.0)
    mean = ssum[:, :_H] * inv
    xr = jnp.dot(x_ref[...], r1_ref[...], preferred_element_type=jnp.float32)
    h = jnp.maximum(mean + xr + b1_ref[...], 0.0)
    h_ref[...] = jnp.concatenate(
        [h, inv, jnp.zeros((_N, _MW - _H - 1), jnp.float32)], axis=1)


def _tc_out_body(q_ref, h_ref, r2_ref, b2_ref, ow_ref, ob_ref, o_ref):
    ssum = q_ref[0] + q_ref[1]
    mean = ssum[:, :_H] * h_ref[:, _H:_H + 1]
    g = mean + jnp.dot(h_ref[:, :_H], r2_ref[...],
                       preferred_element_type=jnp.float32) + b2_ref[...]
    o_ref[...] = jnp.dot(g, ow_ref[...],
                         preferred_element_type=jnp.float32) + ob_ref[...]


def _full(shape):
    return pl.BlockSpec(shape, lambda i: tuple(0 for _ in shape))


def kernel(x, edge_index, edge_attr, nn1_w1, nn1_b1, nn1_w2, nn1_b2, root1,
           bias1, nn2_w1, nn2_b1, nn2_w2, nn2_b2, root2, bias2, out_w, out_b):
    f32 = jnp.float32
    srcw = edge_index[0].reshape(_NW, _PER_W)
    src3 = jnp.concatenate(
        [srcw, jnp.zeros((_NW, _GNC * _GCH - _PER_W), jnp.int32)],
        axis=1).reshape(_NW, _GNC, _GCH)
    dstw = edge_index[1].reshape(_NW, _PER_W)
    dst3 = jnp.concatenate(
        [dstw, jnp.full((_NW, _SNC * _SCH - _PER_W), _N, jnp.int32)],
        axis=1).reshape(_NW, _SNC, _SCH)

    w2t1 = nn1_w2.reshape(128, _IN, _H).transpose(0, 2, 1).reshape(128, _H * _IN)
    b2t1 = nn1_b2.reshape(_IN, _H).T.reshape(1, _H * _IN)
    w2t2 = nn2_w2.reshape(128, _H, _H).transpose(0, 2, 1).reshape(128, _H * _H)
    b2t2 = nn2_b2.reshape(_H, _H).T.reshape(1, _H * _H)
    s1 = jnp.repeat(jnp.eye(_H, dtype=f32), _IN, axis=0)
    s2 = jnp.repeat(jnp.eye(_H, dtype=f32), _H, axis=0)
    b1r = nn1_b1.reshape(1, 128)
    c1r = nn2_b1.reshape(1, 128)
    bias1r = bias1.reshape(1, _H)
    bias2r = bias2.reshape(1, _H)
    out_br = out_b.reshape(1, _OUT)

    zrows = jnp.zeros((_ZCH, _MW), f32)

    xg = _make_sc_gather(_IN)(x, src3)

    grid = (_E // _B,)
    msg1, wm2 = pl.pallas_call(
        _tc_edge1_body,
        grid=grid,
        in_specs=[
            pl.BlockSpec((_B, _ED), lambda i: (i, 0)),
            pl.BlockSpec((_B, _IN), lambda i: (i, 0)),
            _full((_ED, 128)), _full((1, 128)),
            _full((128, _H * _IN)), _full((1, _H * _IN)),
            _full((_ED, 128)), _full((1, 128)),
            _full((128, _H * _H)), _full((1, _H * _H)),
            _full((_H * _IN, _H)),
        ],
        out_specs=[pl.BlockSpec((_B, _MW), lambda i: (i, 0)),
                   pl.BlockSpec((_B, _H * _H), lambda i: (i, 0))],
        out_shape=[jax.ShapeDtypeStruct((_E, _MW), f32),
                   jax.ShapeDtypeStruct((_E, _H * _H), f32)],
    )(edge_attr, xg, nn1_w1, b1r, w2t1, b2t1, nn2_w1, c1r, w2t2, b2t2, s1)

    p1 = _sc_scatter_add(msg1, dst3, zrows)

    hpad = pl.pallas_call(
        _tc_node1_body,
        out_shape=jax.ShapeDtypeStruct((_N, _MW), f32),
    )(p1, x, root1, bias1r)

    hg = _make_sc_gather(_MW)(hpad, src3)

    msg2 = pl.pallas_call(
        _tc_edge2_body,
        grid=grid,
        in_specs=[
            pl.BlockSpec((_B, _MW), lambda i: (i, 0)),
            pl.BlockSpec((_B, _H * _H), lambda i: (i, 0)),
            _full((_H * _H, _H)),
        ],
        out_specs=pl.BlockSpec((_B, _MW), lambda i: (i, 0)),
        out_shape=jax.ShapeDtypeStruct((_E, _MW), f32),
    )(hg, wm2, s2)

    p2 = _sc_scatter_add(msg2, dst3, zrows)

    out = pl.pallas_call(
        _tc_out_body,
        out_shape=jax.ShapeDtypeStruct((_N, _OUT), f32),
    )(p2, hpad, root2, bias2r, out_w, out_br)
    return out
